# baseline (speedup 1.0000x reference)
import jax
import jax.numpy as jnp
from jax.experimental import pallas as pl
from jax.experimental.pallas import tpu as pltpu


def _body(adj0_ref, adj1_ref, x_ref, w_ref, out_ref):
    bm = adj0_ref.shape[0]
    ax0 = jnp.dot(adj0_ref[:], x_ref[:], preferred_element_type=jnp.float32)
    out_ref[:bm, :] = jnp.dot(ax0, w_ref[:], preferred_element_type=jnp.float32)
    ax1 = jnp.dot(adj1_ref[:], x_ref[:], preferred_element_type=jnp.float32)
    out_ref[bm:, :] = jnp.dot(ax1, w_ref[:], preferred_element_type=jnp.float32)


def kernel(x, adj, w):
    n, in_dim = x.shape
    out_dim = w.shape[1]

    bm = 200
    grid = (n // (2 * bm),)
    out = pl.pallas_call(
        _body,
        grid=grid,
        in_specs=[
            pl.BlockSpec((bm, n), lambda i: (2 * i, 0)),
            pl.BlockSpec((bm, n), lambda i: (2 * i + 1, 0)),
            pl.BlockSpec((n, in_dim), lambda i: (0, 0)),
            pl.BlockSpec((in_dim, out_dim), lambda i: (0, 0)),
        ],
        out_specs=pl.BlockSpec((2 * bm, out_dim), lambda i: (i, 0)),
        out_shape=jax.ShapeDtypeStruct((n, out_dim), jnp.float32),
        compiler_params=pltpu.CompilerParams(
            dimension_semantics=("parallel",),
        ),
    )(adj, adj, x, w)
    return out


# speedup vs baseline: 1.0864x; 1.0864x over previous
import jax
import jax.numpy as jnp
from jax.experimental import pallas as pl
from jax.experimental.pallas import tpu as pltpu


def _body(adj_ref, x_ref, w_ref, out_ref):
    ax = jnp.dot(adj_ref[:], x_ref[:], preferred_element_type=jnp.float32)
    out_ref[:] = jnp.dot(ax, w_ref[:], preferred_element_type=jnp.float32)


def kernel(x, adj, w):
    n, in_dim = x.shape
    out_dim = w.shape[1]

    bm = 400
    grid = (n // bm,)
    out = pl.pallas_call(
        _body,
        grid=grid,
        in_specs=[
            pl.BlockSpec((bm, n), lambda i: (i, 0)),
            pl.BlockSpec((n, in_dim), lambda i: (0, 0)),
            pl.BlockSpec((in_dim, out_dim), lambda i: (0, 0)),
        ],
        out_specs=pl.BlockSpec((bm, out_dim), lambda i: (i, 0)),
        out_shape=jax.ShapeDtypeStruct((n, out_dim), jnp.float32),
        compiler_params=pltpu.CompilerParams(
            dimension_semantics=("parallel",),
        ),
    )(adj, x, w)
    return out
